# baseline (speedup 1.0000x reference)
import functools

import jax
import jax.numpy as jnp
import numpy as np
from jax.experimental import pallas as pl
from jax.experimental.pallas import tpu as pltpu

_K = 8
_EPS = float(np.finfo(np.float32).tiny)
_SHAPE = (128, 32768)


def _rotl(x, r):
    return (x << np.uint32(r)) | (x >> np.uint32(32 - r))


def _threefry2x32(k0, k1, c0, c1):
    ks = [np.uint32(k0), np.uint32(k1),
          np.uint32(np.uint32(k0) ^ np.uint32(k1) ^ np.uint32(0x1BD11BDA))]
    x0 = c0 + ks[0]
    x1 = c1 + ks[1]
    rotations = [(13, 15, 26, 6), (17, 29, 16, 24)]
    for i in range(5):
        for r in rotations[i % 2]:
            x0 = x0 + x1
            x1 = _rotl(x1, r)
            x1 = x1 ^ x0
        x0 = x0 + ks[(i + 1) % 3]
        x1 = x1 + ks[(i + 2) % 3] + np.uint32(i + 1)
    return x0, x1


def _gumbel_key1(shape):
    n = int(np.prod(shape))
    b1, b2 = _threefry2x32(0, 1, np.zeros(n, np.uint32),
                           np.arange(n, dtype=np.uint32))
    bits = b1 ^ b2
    fb = (bits >> np.uint32(9)) | np.uint32(0x3F800000)
    f = fb.view(np.float32) - np.float32(1.0)
    tiny = np.float32(np.finfo(np.float32).tiny)
    u = np.maximum(tiny, f * (np.float32(1.0) - tiny) + tiny)
    return (-np.log(-np.log(u))).astype(np.float32).reshape(shape)


_GUMBEL = _gumbel_key1(_SHAPE)


def _subset_body(s_ref, g_ref, o_ref):
    s = s_ref[...] + g_ref[...]
    m = jnp.max(s, axis=1, keepdims=True)
    w = jnp.exp(s - m)
    khot = jnp.zeros_like(w)
    for _ in range(_K):
        z = jnp.sum(w, axis=1, keepdims=True)
        p = w * (1.0 / z)
        khot = khot + p
        w = w - p * w
    o_ref[...] = khot


@jax.jit
def kernel(scores):
    rows, cols = scores.shape
    g = jnp.asarray(_gumbel_key1(scores.shape) if scores.shape != _SHAPE
                    else _GUMBEL, dtype=scores.dtype)
    br = 32 if rows % 32 == 0 else rows
    grid = (rows // br,)
    spec = pl.BlockSpec((br, cols), lambda i: (i, 0))
    return pl.pallas_call(
        _subset_body,
        grid=grid,
        in_specs=[spec, spec],
        out_specs=spec,
        out_shape=jax.ShapeDtypeStruct((rows, cols), scores.dtype),
        compiler_params=pltpu.CompilerParams(
            dimension_semantics=("parallel",),
        ),
    )(scores, g)


# speedup vs baseline: 1.0109x; 1.0109x over previous
import functools

import jax
import jax.numpy as jnp
import numpy as np
from jax.experimental import pallas as pl
from jax.experimental.pallas import tpu as pltpu

_K = 8
_EPS = float(np.finfo(np.float32).tiny)
_SHAPE = (128, 32768)


def _rotl(x, r):
    return (x << np.uint32(r)) | (x >> np.uint32(32 - r))


def _threefry2x32(k0, k1, c0, c1):
    ks = [np.uint32(k0), np.uint32(k1),
          np.uint32(np.uint32(k0) ^ np.uint32(k1) ^ np.uint32(0x1BD11BDA))]
    x0 = c0 + ks[0]
    x1 = c1 + ks[1]
    rotations = [(13, 15, 26, 6), (17, 29, 16, 24)]
    for i in range(5):
        for r in rotations[i % 2]:
            x0 = x0 + x1
            x1 = _rotl(x1, r)
            x1 = x1 ^ x0
        x0 = x0 + ks[(i + 1) % 3]
        x1 = x1 + ks[(i + 2) % 3] + np.uint32(i + 1)
    return x0, x1


def _gumbel_key1(shape):
    n = int(np.prod(shape))
    b1, b2 = _threefry2x32(0, 1, np.zeros(n, np.uint32),
                           np.arange(n, dtype=np.uint32))
    bits = b1 ^ b2
    fb = (bits >> np.uint32(9)) | np.uint32(0x3F800000)
    f = fb.view(np.float32) - np.float32(1.0)
    tiny = np.float32(np.finfo(np.float32).tiny)
    u = np.maximum(tiny, f * (np.float32(1.0) - tiny) + tiny)
    return (-np.log(-np.log(u))).astype(np.float32).reshape(shape)


_GUMBEL = _gumbel_key1(_SHAPE)


def _subset_body(s_ref, g_ref, o_ref):
    s = s_ref[...] + g_ref[...]
    m = jnp.max(s, axis=1, keepdims=True)
    w = jnp.exp(s - m)
    z = jnp.sum(w, axis=1, keepdims=True)
    p = w * (1.0 / z)
    khot = p
    w = w - p * w
    for _ in range(_K - 2):
        z = jnp.sum(w, axis=1, keepdims=True)
        p = w * (1.0 / z)
        khot = khot + p
        w = w - p * w
    z = jnp.sum(w, axis=1, keepdims=True)
    o_ref[...] = khot + w * (1.0 / z)


@jax.jit
def kernel(scores):
    rows, cols = scores.shape
    g = jnp.asarray(_gumbel_key1(scores.shape) if scores.shape != _SHAPE
                    else _GUMBEL, dtype=scores.dtype)
    br = 16 if rows % 16 == 0 else rows
    grid = (rows // br,)
    spec = pl.BlockSpec((br, cols), lambda i: (i, 0))
    return pl.pallas_call(
        _subset_body,
        grid=grid,
        in_specs=[spec, spec],
        out_specs=spec,
        out_shape=jax.ShapeDtypeStruct((rows, cols), scores.dtype),
        compiler_params=pltpu.CompilerParams(
            dimension_semantics=("arbitrary",),
        ),
    )(scores, g)


# speedup vs baseline: 1.0117x; 1.0007x over previous
import jax
import jax.numpy as jnp
import numpy as np
from jax.experimental import pallas as pl
from jax.experimental.pallas import tpu as pltpu

_K = 8
_SHAPE = (128, 32768)


def _rotl(x, r):
    return (x << np.uint32(r)) | (x >> np.uint32(32 - r))


def _threefry2x32(k0, k1, c0, c1):
    ks = [np.uint32(k0), np.uint32(k1),
          np.uint32(np.uint32(k0) ^ np.uint32(k1) ^ np.uint32(0x1BD11BDA))]
    x0 = c0 + ks[0]
    x1 = c1 + ks[1]
    rotations = [(13, 15, 26, 6), (17, 29, 16, 24)]
    for i in range(5):
        for r in rotations[i % 2]:
            x0 = x0 + x1
            x1 = _rotl(x1, r)
            x1 = x1 ^ x0
        x0 = x0 + ks[(i + 1) % 3]
        x1 = x1 + ks[(i + 2) % 3] + np.uint32(i + 1)
    return x0, x1


def _gumbel_key1(shape):
    n = int(np.prod(shape))
    b1, b2 = _threefry2x32(0, 1, np.zeros(n, np.uint32),
                           np.arange(n, dtype=np.uint32))
    bits = b1 ^ b2
    fb = (bits >> np.uint32(9)) | np.uint32(0x3F800000)
    f = fb.view(np.float32) - np.float32(1.0)
    tiny = np.float32(np.finfo(np.float32).tiny)
    u = np.maximum(tiny, f * (np.float32(1.0) - tiny) + tiny)
    return (-np.log(-np.log(u))).astype(np.float32).reshape(shape)


_GUMBEL = _gumbel_key1(_SHAPE)


def _subset_body(s_ref, g_ref, o_ref):
    s = s_ref[...] + g_ref[...]
    m = jnp.max(s, axis=1, keepdims=True)
    w = jnp.exp(s - m)
    z = jnp.sum(w, axis=1, keepdims=True)
    p = w * (1.0 / z)
    khot = p
    w = w - p * w
    for _ in range(_K - 2):
        z = jnp.sum(w, axis=1, keepdims=True)
        p = w * (1.0 / z)
        khot = khot + p
        w = w - p * w
    z = jnp.sum(w, axis=1, keepdims=True)
    o_ref[...] = khot + w * (1.0 / z)


@jax.jit
def kernel(scores):
    rows, cols = scores.shape
    g = jnp.asarray(_gumbel_key1(scores.shape) if scores.shape != _SHAPE
                    else _GUMBEL, dtype=scores.dtype)
    br = 16 if rows % 16 == 0 else rows
    grid = (rows // br,)
    spec = pl.BlockSpec((br, cols), lambda i: (i, 0))
    return pl.pallas_call(
        _subset_body,
        grid=grid,
        in_specs=[spec, spec],
        out_specs=spec,
        out_shape=jax.ShapeDtypeStruct((rows, cols), scores.dtype),
        compiler_params=pltpu.CompilerParams(
            dimension_semantics=("arbitrary",),
        ),
    )(scores, g)
